# Initial kernel scaffold; baseline (speedup 1.0000x reference)
import functools

import jax
import jax.numpy as jnp
from jax import lax
from jax.experimental import pallas as pl
from jax.experimental.pallas import tpu as pltpu
from jax.experimental.pallas import tpu_sc as plsc

B = 2
C = 128
N = 10000
K = 6
NSLOT = K + 1
NP = 10240
BN = 1024
NC, NS = 2, 16
NTILES = NC * NS
CH = 16
IPC = CH * NSLOT
WPT = (B * NP) // NTILES
CPT = WPT // CH


def _mm_body(x_ref, w_ref, out_ref):
    out_ref[0] = lax.dot_general(
        x_ref[0], w_ref[...],
        (((0,), (0,)), ((), ())),
        preferred_element_type=jnp.float32,
    )


def _tc_matmul(x_pad, w_all):
    return pl.pallas_call(
        _mm_body,
        grid=(B, NP // BN),
        in_specs=[
            pl.BlockSpec((1, C, BN), lambda b, j: (b, 0, j)),
            pl.BlockSpec((C, NSLOT * C), lambda b, j: (0, 0)),
        ],
        out_specs=pl.BlockSpec((1, BN, NSLOT * C), lambda b, j: (b, j, 0)),
        out_shape=jax.ShapeDtypeStruct((B, NP, NSLOT * C), jnp.float32),
    )(x_pad, w_all)


_mesh = plsc.VectorSubcoreMesh(core_axis_name="c", subcore_axis_name="s")


@functools.partial(
    pl.kernel,
    out_type=jax.ShapeDtypeStruct((B * NP, C), jnp.float32),
    mesh=_mesh,
    scratch_types=[
        pltpu.VMEM((CPT, IPC), jnp.int32),
        pltpu.VMEM((WPT, 16), jnp.float32),
        pltpu.VMEM((C,), jnp.float32),
        pltpu.VMEM((2, IPC, C), jnp.float32),
        pltpu.VMEM((2, CH, C), jnp.float32),
        pltpu.SemaphoreType.DMA,
        pltpu.SemaphoreType.DMA,
    ],
)
def _sc_gather_sum(z_hbm, idx_hbm, recip_hbm, bias_hbm, out_hbm,
                   idx_v, recip_v, bias_v, gbuf, obuf, sem0, sem1):
    wid = lax.axis_index("s") * NC + lax.axis_index("c")
    base_w = wid * WPT
    base_c = wid * CPT
    pltpu.sync_copy(idx_hbm.at[pl.ds(base_c, CPT)], idx_v)
    pltpu.sync_copy(recip_hbm.at[pl.ds(base_w, WPT)], recip_v)
    pltpu.sync_copy(bias_hbm, bias_v)
    sems = (sem0, sem1)
    pltpu.async_copy(z_hbm.at[idx_v.at[0]], gbuf.at[0], sem0)
    pltpu.async_copy(z_hbm.at[idx_v.at[1]], gbuf.at[1], sem1)

    @pl.loop(0, CPT, step=2)
    def _outer(g0):
        for bb in range(2):
            g = g0 + bb
            pltpu.make_async_copy(
                z_hbm.at[idx_v.at[g]], gbuf.at[bb], sems[bb]).wait()

            @pl.loop(0, CH)
            def _node(i):
                r = i * NSLOT
                for c in range(C // 16):
                    sl = pl.ds(c * 16, 16)
                    acc = gbuf[bb, r, sl]
                    for s in range(1, NSLOT):
                        acc = acc + gbuf[bb, r + s, sl]
                    obuf[bb, i, sl] = acc * recip_v[g * CH + i] + bias_v[sl]

            pltpu.sync_copy(obuf.at[bb],
                            out_hbm.at[pl.ds(base_w + g * CH, CH)])

            @pl.when(g + 2 < CPT)
            def _prefetch():
                pltpu.async_copy(
                    z_hbm.at[idx_v.at[g + 2]], gbuf.at[bb], sems[bb])


def kernel(x, weight_center, weight_neighbors, bias, neighbors):
    x_pad = jnp.pad(x, ((0, 0), (0, 0), (0, NP - N)))
    w_stack = jnp.concatenate(
        [weight_center[None], jnp.moveaxis(weight_neighbors, 2, 0)], axis=0)
    w_all = jnp.transpose(w_stack, (2, 0, 1)).reshape(C, NSLOT * C)

    valid = neighbors >= 0
    safe = jnp.where(valid, neighbors, N).astype(jnp.int32)
    recip = 1.0 / (valid.sum(axis=1).astype(jnp.float32) + 1.0)

    node_ids = jnp.arange(N, dtype=jnp.int32)[:, None]
    slot_ids = jnp.arange(NSLOT, dtype=jnp.int32)[None, :]
    per_node = jnp.concatenate([node_ids, safe], axis=1) * NSLOT + slot_ids
    pad_rows = jnp.full((NP - N, NSLOT), N * NSLOT, dtype=jnp.int32)
    rows_p = jnp.concatenate([per_node, pad_rows], axis=0)
    boff = (jnp.arange(B, dtype=jnp.int32) * (NP * NSLOT))[:, None, None]
    idx_all = (rows_p[None] + boff).reshape((B * NP) // CH, IPC)

    recip_p = jnp.concatenate([recip, jnp.zeros((NP - N,), jnp.float32)])
    recip_splat = jnp.broadcast_to(
        jnp.tile(recip_p, (B,))[:, None], (B * NP, 16))

    z = _tc_matmul(x_pad, w_all)
    z_flat = z.reshape(B * NP * NSLOT, C)

    out_rows = _sc_gather_sum(z_flat, idx_all, recip_splat,
                              bias.astype(jnp.float32))

    out = out_rows.reshape(B, NP, C)[:, :N, :]
    return jnp.transpose(out, (0, 2, 1))


# speedup vs baseline: 2.1238x; 2.1238x over previous
import functools

import jax
import jax.numpy as jnp
from jax import lax
from jax.experimental import pallas as pl
from jax.experimental.pallas import tpu as pltpu
from jax.experimental.pallas import tpu_sc as plsc

B = 2
C = 128
N = 10000
K = 6
NSLOT = K + 1
NP = 10240
BN = 1024
NC, NS = 2, 16
NTILES = NC * NS
CH = 16
IPC = CH * NSLOT
WPT = (B * NP) // NTILES
CPT = WPT // CH


def _mm_body(x_ref, w_ref, out_ref):
    out_ref[0] = lax.dot_general(
        x_ref[0], w_ref[...],
        (((0,), (0,)), ((), ())),
        preferred_element_type=jnp.float32,
    )


def _tc_matmul(x_pad, w_all):
    return pl.pallas_call(
        _mm_body,
        grid=(B, NP // BN),
        in_specs=[
            pl.BlockSpec((1, C, BN), lambda b, j: (b, 0, j)),
            pl.BlockSpec((C, NSLOT * C), lambda b, j: (0, 0)),
        ],
        out_specs=pl.BlockSpec((1, BN, NSLOT * C), lambda b, j: (b, j, 0)),
        out_shape=jax.ShapeDtypeStruct((B, NP, NSLOT * C), jnp.float32),
    )(x_pad, w_all)


@functools.cache
def _make_sc_gather_sum():
    mesh = plsc.VectorSubcoreMesh(
        core_axis_name="c", subcore_axis_name="s",
        num_cores=NC, num_subcores=NS)
    return pl.kernel(
        _sc_body,
        out_type=jax.ShapeDtypeStruct((B * NP, C), jnp.float32),
        mesh=mesh,
        scratch_types=[
            pltpu.VMEM((CPT, IPC), jnp.int32),
            pltpu.VMEM((WPT, 16), jnp.float32),
            pltpu.VMEM((C,), jnp.float32),
            pltpu.VMEM((2, IPC, C), jnp.float32),
            pltpu.VMEM((2, CH, C), jnp.float32),
            pltpu.SemaphoreType.DMA,
            pltpu.SemaphoreType.DMA,
        ],
    )


def _sc_body(z_hbm, idx_hbm, recip_hbm, bias_hbm, out_hbm,
                   idx_v, recip_v, bias_v, gbuf, obuf, sem0, sem1):
    wid = lax.axis_index("s") * NC + lax.axis_index("c")
    base_w = wid * WPT
    base_c = wid * CPT
    pltpu.sync_copy(idx_hbm.at[pl.ds(base_c, CPT)], idx_v)
    pltpu.sync_copy(recip_hbm.at[pl.ds(base_w, WPT)], recip_v)
    pltpu.sync_copy(bias_hbm, bias_v)
    sems = (sem0, sem1)
    pltpu.async_copy(z_hbm.at[idx_v.at[0]], gbuf.at[0], sem0)
    pltpu.async_copy(z_hbm.at[idx_v.at[1]], gbuf.at[1], sem1)

    @pl.loop(0, CPT, step=2)
    def _outer(g0):
        for bb in range(2):
            g = g0 + bb
            pltpu.make_async_copy(
                z_hbm.at[idx_v.at[g]], gbuf.at[bb], sems[bb]).wait()

            @pl.loop(0, CH)
            def _node(i):
                r = i * NSLOT
                for c in range(C // 16):
                    sl = pl.ds(c * 16, 16)
                    acc = gbuf[bb, r, sl]
                    for s in range(1, NSLOT):
                        acc = acc + gbuf[bb, r + s, sl]
                    obuf[bb, i, sl] = acc * recip_v[g * CH + i] + bias_v[sl]

            pltpu.sync_copy(obuf.at[bb],
                            out_hbm.at[pl.ds(base_w + g * CH, CH)])

            @pl.when(g + 2 < CPT)
            def _prefetch():
                pltpu.async_copy(
                    z_hbm.at[idx_v.at[g + 2]], gbuf.at[bb], sems[bb])


def kernel(x, weight_center, weight_neighbors, bias, neighbors):
    x_pad = jnp.pad(x, ((0, 0), (0, 0), (0, NP - N)))
    w_stack = jnp.concatenate(
        [weight_center[None], jnp.moveaxis(weight_neighbors, 2, 0)], axis=0)
    w_all = jnp.transpose(w_stack, (2, 0, 1)).reshape(C, NSLOT * C)

    valid = neighbors >= 0
    safe = jnp.where(valid, neighbors, N).astype(jnp.int32)
    recip = 1.0 / (valid.sum(axis=1).astype(jnp.float32) + 1.0)

    node_ids = jnp.arange(N, dtype=jnp.int32)[:, None]
    slot_ids = jnp.arange(NSLOT, dtype=jnp.int32)[None, :]
    per_node = jnp.concatenate([node_ids, safe], axis=1) * NSLOT + slot_ids
    pad_rows = jnp.full((NP - N, NSLOT), N * NSLOT, dtype=jnp.int32)
    rows_p = jnp.concatenate([per_node, pad_rows], axis=0)
    boff = (jnp.arange(B, dtype=jnp.int32) * (NP * NSLOT))[:, None, None]
    idx_all = (rows_p[None] + boff).reshape((B * NP) // CH, IPC)

    recip_p = jnp.concatenate([recip, jnp.zeros((NP - N,), jnp.float32)])
    recip_splat = jnp.broadcast_to(
        jnp.tile(recip_p, (B,))[:, None], (B * NP, 16))

    z = _tc_matmul(x_pad, w_all)
    z_flat = z.reshape(B * NP * NSLOT, C)

    out_rows = _make_sc_gather_sum()(z_flat, idx_all, recip_splat,
                                     bias.astype(jnp.float32))

    out = out_rows.reshape(B, NP, C)[:, :N, :]
    return jnp.transpose(out, (0, 2, 1))


# speedup vs baseline: 2.1639x; 1.0189x over previous
import functools

import jax
import jax.numpy as jnp
from jax import lax
from jax.experimental import pallas as pl
from jax.experimental.pallas import tpu as pltpu
from jax.experimental.pallas import tpu_sc as plsc

B = 2
C = 128
N = 10000
K = 6
NSLOT = K + 1
NP = 10240
BN = 1024
NC, NS = 2, 16
NTILES = NC * NS
CH = 8
IPC = CH * NSLOT
WPT = (B * NP) // NTILES
CPT = WPT // CH
NBUF = 4


def _mm_body(x_ref, w_ref, out_ref):
    out_ref[0] = lax.dot_general(
        x_ref[0], w_ref[...],
        (((0,), (0,)), ((), ())),
        preferred_element_type=jnp.float32,
    )


def _tc_matmul(x_pad, w_all):
    return pl.pallas_call(
        _mm_body,
        grid=(B, NP // BN),
        in_specs=[
            pl.BlockSpec((1, C, BN), lambda b, j: (b, 0, j)),
            pl.BlockSpec((C, NSLOT * C), lambda b, j: (0, 0)),
        ],
        out_specs=pl.BlockSpec((1, BN, NSLOT * C), lambda b, j: (b, j, 0)),
        out_shape=jax.ShapeDtypeStruct((B, NP, NSLOT * C), jnp.float32),
    )(x_pad, w_all)


@functools.cache
def _make_sc_gather_sum():
    mesh = plsc.VectorSubcoreMesh(
        core_axis_name="c", subcore_axis_name="s",
        num_cores=NC, num_subcores=NS)
    return pl.kernel(
        _sc_body,
        out_type=jax.ShapeDtypeStruct((B * NP, C), jnp.float32),
        mesh=mesh,
        scratch_types=[
            pltpu.VMEM((CPT, IPC), jnp.int32),
            pltpu.VMEM((WPT, 16), jnp.float32),
            pltpu.VMEM((C,), jnp.float32),
            pltpu.VMEM((NBUF, IPC, C), jnp.float32),
            pltpu.VMEM((NBUF, CH, C), jnp.float32),
            [pltpu.SemaphoreType.DMA] * NBUF,
            [pltpu.SemaphoreType.DMA] * NBUF,
        ],
    )


def _sc_body(z_hbm, idx_hbm, recip_hbm, bias_hbm, out_hbm,
             idx_v, recip_v, bias_v, gbuf, obuf, gsems, osems):
    wid = lax.axis_index("s") * NC + lax.axis_index("c")
    base_w = wid * WPT
    base_c = wid * CPT
    pltpu.sync_copy(idx_hbm.at[pl.ds(base_c, CPT)], idx_v)
    pltpu.sync_copy(recip_hbm.at[pl.ds(base_w, WPT)], recip_v)
    pltpu.sync_copy(bias_hbm, bias_v)
    for bb in range(NBUF):
        pltpu.async_copy(z_hbm.at[idx_v.at[bb]], gbuf.at[bb], gsems[bb])

    @pl.loop(0, CPT, step=NBUF)
    def _outer(g0):
        for bb in range(NBUF):
            g = g0 + bb
            pltpu.make_async_copy(
                z_hbm.at[idx_v.at[g]], gbuf.at[bb], gsems[bb]).wait()

            @pl.when(g >= NBUF)
            def _drain():
                pltpu.make_async_copy(
                    obuf.at[bb], out_hbm.at[pl.ds(base_w, CH)],
                    osems[bb]).wait()

            @pl.loop(0, CH)
            def _node(i):
                r = i * NSLOT
                rcp = recip_v[g * CH + i]
                for c in range(C // 16):
                    sl = pl.ds(c * 16, 16)
                    acc = gbuf[bb, r, sl]
                    for s in range(1, NSLOT):
                        acc = acc + gbuf[bb, r + s, sl]
                    obuf[bb, i, sl] = acc * rcp + bias_v[sl]

            pltpu.async_copy(obuf.at[bb],
                             out_hbm.at[pl.ds(base_w + g * CH, CH)],
                             osems[bb])

            @pl.when(g + NBUF < CPT)
            def _prefetch():
                pltpu.async_copy(
                    z_hbm.at[idx_v.at[g + NBUF]], gbuf.at[bb], gsems[bb])

    for bb in range(NBUF):
        pltpu.make_async_copy(
            obuf.at[bb], out_hbm.at[pl.ds(base_w, CH)], osems[bb]).wait()


def kernel(x, weight_center, weight_neighbors, bias, neighbors):
    x_pad = jnp.pad(x, ((0, 0), (0, 0), (0, NP - N)))
    w_stack = jnp.concatenate(
        [weight_center[None], jnp.moveaxis(weight_neighbors, 2, 0)], axis=0)
    w_all = jnp.transpose(w_stack, (2, 0, 1)).reshape(C, NSLOT * C)

    valid = neighbors >= 0
    safe = jnp.where(valid, neighbors, N).astype(jnp.int32)
    recip = 1.0 / (valid.sum(axis=1).astype(jnp.float32) + 1.0)

    node_ids = jnp.arange(N, dtype=jnp.int32)[:, None]
    slot_ids = jnp.arange(NSLOT, dtype=jnp.int32)[None, :]
    per_node = jnp.concatenate([node_ids, safe], axis=1) * NSLOT + slot_ids
    pad_rows = jnp.full((NP - N, NSLOT), N * NSLOT, dtype=jnp.int32)
    rows_p = jnp.concatenate([per_node, pad_rows], axis=0)
    boff = (jnp.arange(B, dtype=jnp.int32) * (NP * NSLOT))[:, None, None]
    idx_all = (rows_p[None] + boff).reshape((B * NP) // CH, IPC)

    recip_p = jnp.concatenate([recip, jnp.zeros((NP - N,), jnp.float32)])
    recip_splat = jnp.broadcast_to(
        jnp.tile(recip_p, (B,))[:, None], (B * NP, 16))

    z = _tc_matmul(x_pad, w_all)
    z_flat = z.reshape(B * NP * NSLOT, C)

    out_rows = _make_sc_gather_sum()(z_flat, idx_all, recip_splat,
                                     bias.astype(jnp.float32))

    out = out_rows.reshape(B, NP, C)[:, :N, :]
    return jnp.transpose(out, (0, 2, 1))
